# baseline (speedup 1.0000x reference)
import functools

import jax
import jax.numpy as jnp
from jax import lax
from jax.experimental import pallas as pl
from jax.experimental.pallas import tpu as pltpu
from jax.experimental.pallas import tpu_sc as plsc

P = 4096
D = 4096
K = 8
NC = 2
NS = 16
L = 16
NW = NC * NS
ROWS_PER_W = P // NW
R_TILE = 16
CHUNKS = D // L


def _top8_sum(ms):
    lane_iota = lax.iota(jnp.int32, L)
    total = jnp.float32(0.0)
    ms = list(ms)
    for _ in range(K):
        head = ms[0]
        m = jnp.max(head)
        total = total + m
        first = plsc.all_reduce_ffs(head == m)
        lane = lane_iota == first
        for i in range(K - 1):
            ms[i] = jnp.where(lane, ms[i + 1], ms[i])
        ms[K - 1] = jnp.where(lane, jnp.zeros((L,), jnp.float32), ms[K - 1])
    return total


_SORT8 = ((0, 1), (2, 3), (4, 5), (6, 7),
          (0, 2), (1, 3), (4, 6), (5, 7),
          (1, 2), (5, 6),
          (0, 4), (1, 5), (2, 6), (3, 7),
          (2, 4), (3, 5),
          (1, 2), (3, 4), (5, 6))


def _sort8_desc(vs):
    vs = list(vs)
    for a, b in _SORT8:
        hi = jnp.maximum(vs[a], vs[b])
        lo = jnp.minimum(vs[a], vs[b])
        vs[a], vs[b] = hi, lo
    return vs


def _merge_top8(ms, bs):
    c = [jnp.maximum(ms[i], bs[7 - i]) for i in range(8)]
    for dist in (4, 2, 1):
        for base in range(0, 8, 2 * dist):
            for i in range(base, base + dist):
                hi = jnp.maximum(c[i], c[i + dist])
                lo = jnp.minimum(c[i], c[i + dist])
                c[i], c[i + dist] = hi, lo
    return c


def _tile_result(buf):
    zero = jnp.zeros((L,), jnp.float32)

    def sum_body(j, ss):
        ss = list(ss)
        for r in range(R_TILE):
            ss[r] = ss[r] + jnp.abs(buf[r, pl.ds(j * L, L)])
        return tuple(ss)

    def topk_body(j, cs):
        cs = list(cs)
        for r in range(R_TILE):
            v = jnp.abs(buf[r, pl.ds(j * L, L)])
            cs[r] = jnp.sort(jnp.minimum(cs[r], -jnp.sort(v)))
        return tuple(cs)

    ss = lax.fori_loop(0, CHUNKS, sum_body, (zero,) * R_TILE, unroll=4)
    cs = lax.fori_loop(0, CHUNKS, topk_body, (zero,) * R_TILE, unroll=1)

    keep = lax.iota(jnp.int32, L) < K
    total = jnp.float32(0.0)
    for r in range(R_TILE):
        top8_sum = -jnp.sum(jnp.where(keep, cs[r], zero))
        row_sum = jnp.sum(ss[r])
        total = total + (top8_sum * (1.0 / K) - row_sum * (1.0 / D))
    return total


def _sc_kernel(x_hbm, out_hbm, buf, acc_vmem):
    wid = lax.axis_index("s") * NC + lax.axis_index("c")
    base = wid * ROWS_PER_W

    def tile_body(t, acc):
        pltpu.sync_copy(x_hbm.at[pl.ds(base + t * R_TILE, R_TILE), :], buf)
        return acc + _tile_result(buf)

    acc = lax.fori_loop(0, ROWS_PER_W // R_TILE, tile_body,
                        jnp.float32(0.0))
    acc_vmem[...] = jnp.zeros((L,), jnp.float32) + acc
    pltpu.sync_copy(acc_vmem, out_hbm.at[wid])


@jax.jit
def _run(x):
    mesh = plsc.VectorSubcoreMesh(core_axis_name="c", subcore_axis_name="s")
    partials = pl.kernel(
        _sc_kernel,
        out_type=jax.ShapeDtypeStruct((NW, L), jnp.float32),
        mesh=mesh,
        scratch_types=[
            pltpu.VMEM((R_TILE, D), jnp.float32),
            pltpu.VMEM((L,), jnp.float32),
        ],
        compiler_params=pltpu.CompilerParams(needs_layout_passes=False),
    )(x)
    return jnp.sum(partials[:, 0]) * (1.0 / P)


def kernel(model):
    x = model.reshape(P, D)
    return _run(x)


# speedup vs baseline: 1.3828x; 1.3828x over previous
import functools

import jax
import jax.numpy as jnp
from jax import lax
from jax.experimental import pallas as pl
from jax.experimental.pallas import tpu as pltpu
from jax.experimental.pallas import tpu_sc as plsc

P = 4096
D = 4096
K = 8
NC = 2
NS = 16
L = 16
NW = NC * NS
ROWS_PER_W = P // NW
R_TILE = 8
CHUNKS = D // L


def _top8_sum(ms):
    lane_iota = lax.iota(jnp.int32, L)
    total = jnp.float32(0.0)
    ms = list(ms)
    for _ in range(K):
        head = ms[0]
        m = jnp.max(head)
        total = total + m
        first = plsc.all_reduce_ffs(head == m)
        lane = lane_iota == first
        for i in range(K - 1):
            ms[i] = jnp.where(lane, ms[i + 1], ms[i])
        ms[K - 1] = jnp.where(lane, jnp.zeros((L,), jnp.float32), ms[K - 1])
    return total


_SORT8 = ((0, 1), (2, 3), (4, 5), (6, 7),
          (0, 2), (1, 3), (4, 6), (5, 7),
          (1, 2), (5, 6),
          (0, 4), (1, 5), (2, 6), (3, 7),
          (2, 4), (3, 5),
          (1, 2), (3, 4), (5, 6))


def _sort8_desc(vs):
    vs = list(vs)
    for a, b in _SORT8:
        hi = jnp.maximum(vs[a], vs[b])
        lo = jnp.minimum(vs[a], vs[b])
        vs[a], vs[b] = hi, lo
    return vs


def _merge_top8(ms, bs):
    c = [jnp.maximum(ms[i], bs[7 - i]) for i in range(8)]
    for dist in (4, 2, 1):
        for base in range(0, 8, 2 * dist):
            for i in range(base, base + dist):
                hi = jnp.maximum(c[i], c[i + dist])
                lo = jnp.minimum(c[i], c[i + dist])
                c[i], c[i + dist] = hi, lo
    return c


def _tile_result(buf):
    zero = jnp.zeros((L,), jnp.float32)

    def body(j, carry):
        ss = list(carry[:R_TILE])
        cs = list(carry[R_TILE:])
        for r in range(R_TILE):
            v = jnp.abs(buf[r, pl.ds(j * L, L)])
            ss[r] = ss[r] + v
            cs[r] = jnp.sort(jnp.minimum(cs[r], -jnp.sort(v)))
        return (*ss, *cs)

    out = lax.fori_loop(0, CHUNKS, body, (zero,) * (2 * R_TILE), unroll=2)
    ss = out[:R_TILE]
    cs = out[R_TILE:]

    keep = lax.iota(jnp.int32, L) < K
    total = jnp.float32(0.0)
    for r in range(R_TILE):
        top8_sum = -jnp.sum(jnp.where(keep, cs[r], zero))
        row_sum = jnp.sum(ss[r])
        total = total + (top8_sum * (1.0 / K) - row_sum * (1.0 / D))
    return total


def _sc_kernel(x_hbm, out_hbm, buf0, buf1, acc_vmem, sem0, sem1):
    wid = lax.axis_index("s") * NC + lax.axis_index("c")
    base = wid * ROWS_PER_W
    n_tiles = ROWS_PER_W // R_TILE

    def src(t):
        return x_hbm.at[pl.ds(base + t * R_TILE, R_TILE), :]

    def wait(buf, sem):
        pltpu.make_async_copy(src(0), buf, sem).wait()

    pltpu.async_copy(src(0), buf0, sem0)

    def tile_pair(t, acc):
        wait(buf0, sem0)
        pltpu.async_copy(src(2 * t + 1), buf1, sem1)
        acc = acc + _tile_result(buf0)
        wait(buf1, sem1)

        @pl.when(t < n_tiles // 2 - 1)
        def _():
            pltpu.async_copy(src(2 * t + 2), buf0, sem0)

        return acc + _tile_result(buf1)

    acc = lax.fori_loop(0, n_tiles // 2, tile_pair, jnp.float32(0.0))
    acc_vmem[...] = jnp.zeros((L,), jnp.float32) + acc
    pltpu.sync_copy(acc_vmem, out_hbm.at[wid])


@jax.jit
def _run(x):
    mesh = plsc.VectorSubcoreMesh(core_axis_name="c", subcore_axis_name="s")
    partials = pl.kernel(
        _sc_kernel,
        out_type=jax.ShapeDtypeStruct((NW, L), jnp.float32),
        mesh=mesh,
        scratch_types=[
            pltpu.VMEM((R_TILE, D), jnp.float32),
            pltpu.VMEM((R_TILE, D), jnp.float32),
            pltpu.VMEM((L,), jnp.float32),
            pltpu.SemaphoreType.DMA,
            pltpu.SemaphoreType.DMA,
        ],
        compiler_params=pltpu.CompilerParams(needs_layout_passes=False),
    )(x)
    return jnp.sum(partials[:, 0]) * (1.0 / P)


def kernel(model):
    x = model.reshape(P, D)
    return _run(x)
